# baseline (speedup 1.0000x reference)
import functools

import jax
import jax.numpy as jnp
from jax import lax
from jax.experimental import pallas as pl
from jax.experimental.pallas import tpu as pltpu
from jax.experimental.pallas import tpu_sc as plsc

B = 16384
L = 200
V = 7
D = 4
NW = 32
RPW = B // NW
CB = 256
NCHUNK = RPW // CB
NL4 = (V ** 4 + 15) // 16
PAIR_STEP = 4


def _sc_body(xt_hbm, table_hbm, w_hbm, b_hbm, out_hbm,
             xv, lut1, lut2, lut4, outv, tblv, wv, bv, sems):
    nc = 2
    wid = lax.axis_index("s") * nc + lax.axis_index("c")
    base = wid * RPW

    pltpu.sync_copy(table_hbm, tblv)
    pltpu.sync_copy(w_hbm, wv)
    pltpu.sync_copy(b_hbm, bv)

    lane = lax.iota(jnp.int32, 16)
    zero = jnp.zeros((16,), jnp.int32)

    k_ix = jnp.minimum(lane, V - 1)
    v = jnp.zeros((16,), jnp.float32)
    for j in range(D):
        tcol = plsc.load_gather(tblv, [zero + j, k_ix])
        wj = plsc.load_gather(wv, [zero, zero + j])
        v = v + tcol * wj
    lut1[...] = v * (1.0 / L)
    bias = plsc.load_gather(bv, [zero])

    for i in range(4):
        ix = lane + i * 16
        lut2[pl.ds(i * 16, 16)] = (plsc.load_gather(lut1, [ix // V]) +
                                   plsc.load_gather(lut1, [ix % V]))

    def l4_body(i, carry):
        ix = lane + i * 16
        lut4[pl.ds(i * 16, 16)] = (
            plsc.load_gather(lut2, [ix // (V * V)]) +
            plsc.load_gather(lut2, [ix % (V * V)]))
        return carry

    lax.fori_loop(0, NL4, l4_body, 0)

    def make_chunk_compute(buf, chunk):
        def group_body(g, carry):
            b0 = g * 16

            def col_body(i, acc):
                l = i * PAIR_STEP
                a0 = buf[l, pl.ds(b0, 16)]
                a1 = buf[l + 1, pl.ds(b0, 16)]
                a2 = buf[l + 2, pl.ds(b0, 16)]
                a3 = buf[l + 3, pl.ds(b0, 16)]
                ix = ((a0 * V + a1) * V + a2) * V + a3
                return acc + plsc.load_gather(lut4, [ix])

            acc = lax.fori_loop(0, L // PAIR_STEP, col_body,
                                jnp.zeros((16,), jnp.float32), unroll=2)
            z = acc + bias
            outv[pl.ds(chunk * CB + b0, 16)] = 1.0 / (1.0 + jnp.exp(-z))
            return carry
        return group_body

    copies = [None] * NCHUNK
    copies[0] = pltpu.async_copy(
        xt_hbm.at[:, pl.ds(base, CB)], xv.at[0], sems.at[0])
    for c in range(NCHUNK):
        if c + 1 < NCHUNK:
            copies[c + 1] = pltpu.async_copy(
                xt_hbm.at[:, pl.ds(base + (c + 1) * CB, CB)],
                xv.at[(c + 1) % 2], sems.at[(c + 1) % 2])
        copies[c].wait()
        lax.fori_loop(0, CB // 16, make_chunk_compute(xv.at[c % 2], c), 0)

    pltpu.sync_copy(outv, out_hbm.at[pl.ds(base, RPW)])


@functools.partial(
    pl.kernel,
    out_type=jax.ShapeDtypeStruct((B,), jnp.float32),
    mesh=plsc.VectorSubcoreMesh(core_axis_name="c", subcore_axis_name="s"),
    scratch_types=[
        pltpu.VMEM((2, L, CB), jnp.int32),
        pltpu.VMEM((16,), jnp.float32),
        pltpu.VMEM((64,), jnp.float32),
        pltpu.VMEM((NL4 * 16,), jnp.float32),
        pltpu.VMEM((RPW,), jnp.float32),
        pltpu.VMEM((D, V), jnp.float32),
        pltpu.VMEM((1, D), jnp.float32),
        pltpu.VMEM((1,), jnp.float32),
        pltpu.SemaphoreType.DMA((2,)),
    ],
    compiler_params=pltpu.CompilerParams(needs_layout_passes=False),
)
def _shadow_sc(*args):
    _sc_body(*args)


def kernel(x, table, W, b):
    out = _shadow_sc(x.T, table.T, W.T, b)
    return out.reshape(B, 1)


# speedup vs baseline: 1.0292x; 1.0292x over previous
import functools

import jax
import jax.numpy as jnp
from jax import lax
from jax.experimental import pallas as pl
from jax.experimental.pallas import tpu as pltpu
from jax.experimental.pallas import tpu_sc as plsc

B = 16384
L = 200
V = 7
D = 4
NW = 32
RPW = B // NW
CB = 256
NCHUNK = RPW // CB
NL4 = (V ** 4 + 15) // 16
PAIR_STEP = 4


def _sc_body(xt_hbm, params_hbm, out_hbm,
             xv, lut1, lut2, lut4, outv, pv, sems):
    nc = 2
    wid = lax.axis_index("s") * nc + lax.axis_index("c")
    base = wid * RPW

    pltpu.sync_copy(params_hbm, pv)

    lane = lax.iota(jnp.int32, 16)

    k_ix = jnp.minimum(lane, V - 1) * D
    v = jnp.zeros((16,), jnp.float32)
    for j in range(D):
        tcol = plsc.load_gather(pv, [k_ix + j])
        wj = plsc.load_gather(pv, [jnp.full((16,), V * D + j, jnp.int32)])
        v = v + tcol * wj
    lut1[...] = v * (1.0 / L)
    bias = plsc.load_gather(pv, [jnp.full((16,), V * D + D, jnp.int32)])

    for i in range(4):
        ix = lane + i * 16
        lut2[pl.ds(i * 16, 16)] = (plsc.load_gather(lut1, [ix // V]) +
                                   plsc.load_gather(lut1, [ix % V]))

    def l4_body(i, carry):
        ix = lane + i * 16
        lut4[pl.ds(i * 16, 16)] = (
            plsc.load_gather(lut2, [ix // (V * V)]) +
            plsc.load_gather(lut2, [ix % (V * V)]))
        return carry

    lax.fori_loop(0, NL4, l4_body, 0)

    def make_chunk_compute(buf, chunk):
        def group_body(g, carry):
            b0 = g * 16

            def col_body(i, acc):
                l = i * PAIR_STEP
                a0 = buf[l, pl.ds(b0, 16)]
                a1 = buf[l + 1, pl.ds(b0, 16)]
                a2 = buf[l + 2, pl.ds(b0, 16)]
                a3 = buf[l + 3, pl.ds(b0, 16)]
                ix = ((a0 * V + a1) * V + a2) * V + a3
                return acc + plsc.load_gather(lut4, [ix])

            acc = lax.fori_loop(0, L // PAIR_STEP, col_body,
                                jnp.zeros((16,), jnp.float32), unroll=2)
            z = acc + bias
            outv[pl.ds(chunk * CB + b0, 16)] = 1.0 / (1.0 + jnp.exp(-z))
            return carry
        return group_body

    copies = [None] * NCHUNK
    copies[0] = pltpu.async_copy(
        xt_hbm.at[:, pl.ds(base, CB)], xv.at[0], sems.at[0])
    for c in range(NCHUNK):
        if c + 1 < NCHUNK:
            copies[c + 1] = pltpu.async_copy(
                xt_hbm.at[:, pl.ds(base + (c + 1) * CB, CB)],
                xv.at[(c + 1) % 2], sems.at[(c + 1) % 2])
        copies[c].wait()
        lax.fori_loop(0, CB // 16, make_chunk_compute(xv.at[c % 2], c), 0)

    pltpu.sync_copy(outv, out_hbm.at[pl.ds(base, RPW)])


@functools.partial(
    pl.kernel,
    out_type=jax.ShapeDtypeStruct((B,), jnp.float32),
    mesh=plsc.VectorSubcoreMesh(core_axis_name="c", subcore_axis_name="s"),
    scratch_types=[
        pltpu.VMEM((2, L, CB), jnp.int32),
        pltpu.VMEM((16,), jnp.float32),
        pltpu.VMEM((64,), jnp.float32),
        pltpu.VMEM((NL4 * 16,), jnp.float32),
        pltpu.VMEM((RPW,), jnp.float32),
        pltpu.VMEM((48,), jnp.float32),
        pltpu.SemaphoreType.DMA((2,)),
    ],
    compiler_params=pltpu.CompilerParams(needs_layout_passes=False),
)
def _shadow_sc(*args):
    _sc_body(*args)


def kernel(x, table, W, b):
    params = jnp.concatenate([table.reshape(-1), W.reshape(-1), b])
    params = jnp.pad(params, (0, 48 - params.shape[0]))
    out = _shadow_sc(x.T, params)
    return out.reshape(B, 1)


# speedup vs baseline: 1.0958x; 1.0647x over previous
import functools

import jax
import jax.numpy as jnp
from jax import lax
from jax.experimental import pallas as pl
from jax.experimental.pallas import tpu as pltpu
from jax.experimental.pallas import tpu_sc as plsc

B = 16384
L = 200
V = 7
D = 4
NW = 32
RPW = B // NW
CB = 128
NCHUNK = RPW // CB
NL4 = (V ** 4 + 15) // 16
PAIR_STEP = 4


def _sc_body(xt_hbm, params_hbm, out_hbm,
             xv, lut1, lut2, lut4, outv, pv, sems):
    nc = 2
    wid = lax.axis_index("s") * nc + lax.axis_index("c")
    base = wid * RPW

    pltpu.sync_copy(params_hbm, pv)

    lane = lax.iota(jnp.int32, 16)

    k_ix = jnp.minimum(lane, V - 1) * D
    v = jnp.zeros((16,), jnp.float32)
    for j in range(D):
        tcol = plsc.load_gather(pv, [k_ix + j])
        wj = plsc.load_gather(pv, [jnp.full((16,), V * D + j, jnp.int32)])
        v = v + tcol * wj
    lut1[...] = v * (1.0 / L)
    bias = plsc.load_gather(pv, [jnp.full((16,), V * D + D, jnp.int32)])

    for i in range(4):
        ix = lane + i * 16
        lut2[pl.ds(i * 16, 16)] = (plsc.load_gather(lut1, [ix // V]) +
                                   plsc.load_gather(lut1, [ix % V]))

    def l4_body(i, carry):
        ix = lane + i * 16
        lut4[pl.ds(i * 16, 16)] = (
            plsc.load_gather(lut2, [ix // (V * V)]) +
            plsc.load_gather(lut2, [ix % (V * V)]))
        return carry

    lax.fori_loop(0, NL4, l4_body, 0)

    def make_chunk_compute(buf, chunk):
        def group_body(g, carry):
            b0 = g * 16

            def col_body(i, acc):
                l = i * PAIR_STEP
                a0 = buf[l, pl.ds(b0, 16)]
                a1 = buf[l + 1, pl.ds(b0, 16)]
                a2 = buf[l + 2, pl.ds(b0, 16)]
                a3 = buf[l + 3, pl.ds(b0, 16)]
                ix = ((a0 * V + a1) * V + a2) * V + a3
                return acc + plsc.load_gather(lut4, [ix])

            acc = lax.fori_loop(0, L // PAIR_STEP, col_body,
                                jnp.zeros((16,), jnp.float32), unroll=5)
            z = acc + bias
            outv[pl.ds(chunk * CB + b0, 16)] = 1.0 / (1.0 + jnp.exp(-z))
            return carry
        return group_body

    copies = [None] * NCHUNK
    copies[0] = pltpu.async_copy(
        xt_hbm.at[:, pl.ds(base, CB)], xv.at[0], sems.at[0])
    for c in range(NCHUNK):
        if c + 1 < NCHUNK:
            copies[c + 1] = pltpu.async_copy(
                xt_hbm.at[:, pl.ds(base + (c + 1) * CB, CB)],
                xv.at[(c + 1) % 2], sems.at[(c + 1) % 2])
        copies[c].wait()
        lax.fori_loop(0, CB // 16, make_chunk_compute(xv.at[c % 2], c), 0)

    pltpu.sync_copy(outv, out_hbm.at[pl.ds(base, RPW)])


@functools.partial(
    pl.kernel,
    out_type=jax.ShapeDtypeStruct((B,), jnp.float32),
    mesh=plsc.VectorSubcoreMesh(core_axis_name="c", subcore_axis_name="s"),
    scratch_types=[
        pltpu.VMEM((2, L, CB), jnp.int32),
        pltpu.VMEM((16,), jnp.float32),
        pltpu.VMEM((64,), jnp.float32),
        pltpu.VMEM((NL4 * 16,), jnp.float32),
        pltpu.VMEM((RPW,), jnp.float32),
        pltpu.VMEM((48,), jnp.float32),
        pltpu.SemaphoreType.DMA((2,)),
    ],
    compiler_params=pltpu.CompilerParams(
        needs_layout_passes=False, disable_bounds_checks=True),
)
def _shadow_sc(*args):
    _sc_body(*args)


def kernel(x, table, W, b):
    params = jnp.concatenate([table.reshape(-1), W.reshape(-1), b])
    params = jnp.pad(params, (0, 48 - params.shape[0]))
    out = _shadow_sc(x.T, params)
    return out.reshape(B, 1)
